# Initial kernel scaffold; baseline (speedup 1.0000x reference)
import jax
import jax.numpy as jnp
from jax.experimental import pallas as pl


def kernel(x_state, x_task, x_actor, edge_attr_st, W_e, b_e, W1, b1, W2, b2, Wa1, ba1, Wa2, ba2, edge_src_st, edge_dst_st, edge_src_ta, edge_dst_ta):
    raise NotImplementedError("write your pallas kernel here")


# speedup vs baseline: 2.3348x; 2.3348x over previous
import jax
import jax.numpy as jnp
from jax import lax
from jax.experimental import pallas as pl
from jax.experimental.pallas import tpu as pltpu
from jax.experimental.pallas import tpu_sc as plsc

D = 128
N_STATE, N_TASK, N_ACTOR = 50000, 10000, 65536
E1, E2 = 320000, 262144
NC, NS = 2, 16
NW = NC * NS
CH = 128
NG = D // 16

_mesh = plsc.VectorSubcoreMesh(core_axis_name="c", subcore_axis_name="s")


_C1 = E1 // CH
_C1_BASE = _C1 // NW
_C1_REM = _C1 - _C1_BASE * NW
_RA = 640
_RA_LAST = N_TASK - _RA


def _stage_a_body(x_state, a0_h, a1_h, e_src, e_dst, we, be, part,
                  acc, idx_s, idx_d, a0_v, a1_v, rows, wev, bev, sem):
  cid = lax.axis_index("c")
  sid = lax.axis_index("s")
  wid = sid * NC + cid

  zeros16 = jnp.zeros((16,), jnp.float32)

  def _zero_row(i, _):
    for g in range(NG):
      rows[i, pl.ds(g * 16, 16)] = zeros16
    return 0

  lax.fori_loop(0, CH, _zero_row, 0)
  row0 = jnp.minimum(sid * _RA, _RA_LAST)
  for k in range(_RA // CH):
    pltpu.sync_copy(rows, acc.at[pl.ds(row0 + k * CH, CH)])

  pltpu.sync_copy(we, wev)
  pltpu.sync_copy(be, bev)
  w0 = [wev[0, pl.ds(g * 16, 16)] for g in range(NG)]
  w1 = [wev[1, pl.ds(g * 16, 16)] for g in range(NG)]
  b = [bev[pl.ds(g * 16, 16)] for g in range(NG)]

  plsc.subcore_barrier()

  n_chunks = jnp.where(wid < _C1_REM, _C1_BASE + 1, _C1_BASE)

  def _chunk(k, _):
    base = (wid + k * NW) * CH
    pltpu.sync_copy(e_src.at[pl.ds(base, CH)], idx_s)
    pltpu.sync_copy(e_dst.at[pl.ds(base, CH)], idx_d)
    pltpu.sync_copy(a0_h.at[pl.ds(base, CH)], a0_v.at[pl.ds(0, CH)])
    pltpu.sync_copy(a1_h.at[pl.ds(base, CH)], a1_v.at[pl.ds(0, CH)])
    pltpu.async_copy(x_state.at[idx_s], rows, sem).wait()

    def _edge(e, _):
      a0 = jnp.full((16,), a0_v[pl.ds(e, 16)][0], jnp.float32)
      a1 = jnp.full((16,), a1_v[pl.ds(e, 16)][0], jnp.float32)
      for g in range(NG):
        v = rows[e, pl.ds(g * 16, 16)] + a0 * w0[g] + a1 * w1[g] + b[g]
        rows[e, pl.ds(g * 16, 16)] = jnp.maximum(v, 0.0)
      return 0

    lax.fori_loop(0, CH, _edge, 0)
    pltpu.sync_copy(rows, acc.at[idx_d], add=True)
    return 0

  lax.fori_loop(0, n_chunks, _chunk, 0)
  plsc.subcore_barrier()

  pltpu.sync_copy(acc.at[pl.ds(row0, _RA)], part.at[cid, pl.ds(row0, _RA)])


@jax.jit
def _stage_a(x_state, a0_h, a1_h, e_src, e_dst, we, be):
  return pl.kernel(
      _stage_a_body,
      out_type=jax.ShapeDtypeStruct((NC, N_TASK, D), jnp.float32),
      mesh=_mesh,
      scratch_types=[
          pltpu.VMEM_SHARED((N_TASK, D), jnp.float32),
          pltpu.VMEM((CH,), jnp.int32),
          pltpu.VMEM((CH,), jnp.int32),
          pltpu.VMEM((CH + 16,), jnp.float32),
          pltpu.VMEM((CH + 16,), jnp.float32),
          pltpu.VMEM((CH, D), jnp.float32),
          pltpu.VMEM((2, D), jnp.float32),
          pltpu.VMEM((D,), jnp.float32),
          pltpu.SemaphoreType.DMA,
      ],
  )(x_state, a0_h, a1_h, e_src, e_dst, we, be)


_C2 = E2 // CH
_C2_TILE = _C2 // NS
_G_PER_SC = NG // NC
_AR = N_ACTOR // NS
_ZR = 1024


def _stage_c_body(x1b, e_src, e_dst, agg2b,
                  xg, acc, idx_s, idx_d, rows, zbuf, sem):
  cid = lax.axis_index("c")
  sid = lax.axis_index("s")

  zeros16 = jnp.zeros((16,), jnp.float32)

  def _zero_row(i, _):
    zbuf[i, pl.ds(0, 16)] = zeros16
    return 0

  lax.fori_loop(0, _ZR, _zero_row, 0)
  a0 = sid * _AR
  x0 = jnp.minimum(sid * _RA, _RA_LAST)

  for j in range(_G_PER_SC):
    g = cid * _G_PER_SC + j

    for k in range(_AR // _ZR):
      pltpu.sync_copy(zbuf, acc.at[pl.ds(a0 + k * _ZR, _ZR)])
    pltpu.sync_copy(x1b.at[g, pl.ds(x0, _RA)], xg.at[pl.ds(x0, _RA)])
    plsc.subcore_barrier()

    def _chunk(k, _):
      base = (sid + k * NS) * CH
      pltpu.sync_copy(e_src.at[pl.ds(base, CH)], idx_s)
      pltpu.sync_copy(e_dst.at[pl.ds(base, CH)], idx_d)
      pltpu.async_copy(xg.at[idx_s], rows, sem).wait()
      pltpu.sync_copy(rows, acc.at[idx_d], add=True)
      return 0

    lax.fori_loop(0, _C2_TILE, _chunk, 0)
    plsc.subcore_barrier()

    pltpu.sync_copy(acc.at[pl.ds(a0, _AR)], agg2b.at[g, pl.ds(a0, _AR)])
    plsc.subcore_barrier()


@jax.jit
def _stage_c(x1b, e_src, e_dst):
  return pl.kernel(
      _stage_c_body,
      out_type=jax.ShapeDtypeStruct((NG, N_ACTOR, 16), jnp.float32),
      mesh=_mesh,
      scratch_types=[
          pltpu.VMEM_SHARED((N_TASK, 16), jnp.float32),
          pltpu.VMEM_SHARED((N_ACTOR, 16), jnp.float32),
          pltpu.VMEM((CH,), jnp.int32),
          pltpu.VMEM((CH,), jnp.int32),
          pltpu.VMEM((CH, 16), jnp.float32),
          pltpu.VMEM((_ZR, 16), jnp.float32),
          pltpu.SemaphoreType.DMA,
      ],
      compiler_params=pltpu.CompilerParams(use_tc_tiling_on_sc=False),
  )(x1b, e_src, e_dst)


def _mlp1_body(xt, part, w1, b1, w2, b2, out):
  h = xt[...] + part[0] + part[1]
  y = jnp.maximum(jnp.dot(h, w1[...], preferred_element_type=jnp.float32)
                  + b1[...], 0.0)
  x1 = jnp.dot(y, w2[...], preferred_element_type=jnp.float32) + b2[...]
  for g in range(NG):
    out[g] = x1[:, g * 16:(g + 1) * 16]


@jax.jit
def _mlp1(x_task, part, w1, b1, w2, b2):
  blk = 1000
  grid = N_TASK // blk
  return pl.pallas_call(
      _mlp1_body,
      grid=(grid,),
      in_specs=[
          pl.BlockSpec((blk, D), lambda i: (i, 0)),
          pl.BlockSpec((NC, blk, D), lambda i: (0, i, 0)),
          pl.BlockSpec((D, D), lambda i: (0, 0)),
          pl.BlockSpec((1, D), lambda i: (0, 0)),
          pl.BlockSpec((D, D), lambda i: (0, 0)),
          pl.BlockSpec((1, D), lambda i: (0, 0)),
      ],
      out_specs=pl.BlockSpec((NG, blk, 16), lambda i: (0, i, 0)),
      out_shape=jax.ShapeDtypeStruct((NG, N_TASK, 16), jnp.float32),
  )(x_task, part, w1, b1.reshape(1, D), w2, b2.reshape(1, D))


def _mlp2_body(xa, agg, wa1, ba1, wa2, ba2, out):
  h = xa[...] + jnp.concatenate([agg[g] for g in range(NG)], axis=-1)
  y = jnp.maximum(jnp.dot(h, wa1[...], preferred_element_type=jnp.float32)
                  + ba1[...], 0.0)
  out[...] = (jnp.dot(y, wa2[...], preferred_element_type=jnp.float32)
              + ba2[...])


@jax.jit
def _mlp2(x_actor, agg2b, wa1, ba1, wa2, ba2):
  blk = 4096
  grid = N_ACTOR // blk
  return pl.pallas_call(
      _mlp2_body,
      grid=(grid,),
      in_specs=[
          pl.BlockSpec((blk, D), lambda i: (i, 0)),
          pl.BlockSpec((NG, blk, 16), lambda i: (0, i, 0)),
          pl.BlockSpec((D, D), lambda i: (0, 0)),
          pl.BlockSpec((1, D), lambda i: (0, 0)),
          pl.BlockSpec((D, 1), lambda i: (0, 0)),
          pl.BlockSpec((1, 1), lambda i: (0, 0)),
      ],
      out_specs=pl.BlockSpec((blk, 1), lambda i: (i, 0)),
      out_shape=jax.ShapeDtypeStruct((N_ACTOR, 1), jnp.float32),
  )(x_actor, agg2b, wa1, ba1.reshape(1, D), wa2, ba2.reshape(1, 1))


def kernel(x_state, x_task, x_actor, edge_attr_st, W_e, b_e, W1, b1, W2, b2,
           Wa1, ba1, Wa2, ba2, edge_src_st, edge_dst_st, edge_src_ta,
           edge_dst_ta):
  a0 = edge_attr_st[:, 0]
  a1 = edge_attr_st[:, 1]
  part = _stage_a(x_state, a0, a1,
                  edge_src_st.astype(jnp.int32), edge_dst_st.astype(jnp.int32),
                  W_e, b_e)
  x1b = _mlp1(x_task, part, W1, b1, W2, b2)
  agg2b = _stage_c(x1b, edge_src_ta.astype(jnp.int32),
                   edge_dst_ta.astype(jnp.int32))
  logits = _mlp2(x_actor, agg2b, Wa1, ba1, Wa2, ba2)
  return logits.reshape(-1, 64)


# speedup vs baseline: 3.5919x; 1.5384x over previous
import jax
import jax.numpy as jnp
from jax import lax
from jax.experimental import pallas as pl
from jax.experimental.pallas import tpu as pltpu
from jax.experimental.pallas import tpu_sc as plsc

D = 128
N_STATE, N_TASK, N_ACTOR = 50000, 10000, 65536
E1, E2 = 320000, 262144
NC, NS = 2, 16
NW = NC * NS
CH = 128
NG = D // 16

_mesh = plsc.VectorSubcoreMesh(core_axis_name="c", subcore_axis_name="s")


_C1 = E1 // CH
_C1_BASE = _C1 // NW
_C1_REM = _C1 - _C1_BASE * NW
_RA = 640
_RA_LAST = N_TASK - _RA


def _stage_a_body(x_state, a0_h, a1_h, e_src, e_dst, we, be, part,
                  acc, idx_s, idx_d, a0_v, a1_v, rows, wev, bev, sem):
  cid = lax.axis_index("c")
  sid = lax.axis_index("s")
  wid = sid * NC + cid

  zeros16 = jnp.zeros((16,), jnp.float32)

  def _zero_row(i, _):
    for g in range(NG):
      rows[i, pl.ds(g * 16, 16)] = zeros16
    return 0

  lax.fori_loop(0, CH, _zero_row, 0)
  row0 = jnp.minimum(sid * _RA, _RA_LAST)
  for k in range(_RA // CH):
    pltpu.sync_copy(rows, acc.at[pl.ds(row0 + k * CH, CH)])

  pltpu.sync_copy(we, wev)
  pltpu.sync_copy(be, bev)
  w0 = [wev[0, pl.ds(g * 16, 16)] for g in range(NG)]
  w1 = [wev[1, pl.ds(g * 16, 16)] for g in range(NG)]
  b = [bev[pl.ds(g * 16, 16)] for g in range(NG)]

  plsc.subcore_barrier()

  n_chunks = jnp.where(wid < _C1_REM, _C1_BASE + 1, _C1_BASE)

  def _chunk(k, _):
    base = (wid + k * NW) * CH
    pltpu.sync_copy(e_src.at[pl.ds(base, CH)], idx_s)
    pltpu.sync_copy(e_dst.at[pl.ds(base, CH)], idx_d)
    pltpu.sync_copy(a0_h.at[pl.ds(base, CH)], a0_v.at[pl.ds(0, CH)])
    pltpu.sync_copy(a1_h.at[pl.ds(base, CH)], a1_v.at[pl.ds(0, CH)])
    pltpu.async_copy(x_state.at[idx_s], rows, sem).wait()

    def _edge(e, _):
      a0 = jnp.full((16,), a0_v[pl.ds(e, 16)][0], jnp.float32)
      a1 = jnp.full((16,), a1_v[pl.ds(e, 16)][0], jnp.float32)
      for g in range(NG):
        v = rows[e, pl.ds(g * 16, 16)] + a0 * w0[g] + a1 * w1[g] + b[g]
        rows[e, pl.ds(g * 16, 16)] = jnp.maximum(v, 0.0)
      return 0

    lax.fori_loop(0, CH, _edge, 0)
    pltpu.sync_copy(rows, acc.at[idx_d], add=True)
    return 0

  lax.fori_loop(0, n_chunks, _chunk, 0)
  plsc.subcore_barrier()

  pltpu.sync_copy(acc.at[pl.ds(row0, _RA)], part.at[cid, pl.ds(row0, _RA)])


@jax.jit
def _stage_a(x_state, a0_h, a1_h, e_src, e_dst, we, be):
  return pl.kernel(
      _stage_a_body,
      out_type=jax.ShapeDtypeStruct((NC, N_TASK, D), jnp.float32),
      mesh=_mesh,
      scratch_types=[
          pltpu.VMEM_SHARED((N_TASK, D), jnp.float32),
          pltpu.VMEM((CH,), jnp.int32),
          pltpu.VMEM((CH,), jnp.int32),
          pltpu.VMEM((CH + 16,), jnp.float32),
          pltpu.VMEM((CH + 16,), jnp.float32),
          pltpu.VMEM((CH, D), jnp.float32),
          pltpu.VMEM((2, D), jnp.float32),
          pltpu.VMEM((D,), jnp.float32),
          pltpu.SemaphoreType.DMA,
      ],
  )(x_state, a0_h, a1_h, e_src, e_dst, we, be)


_C2 = E2 // CH
_C2_TILE = _C2 // NS
_G_PER_SC = NG // NC
_AR = N_ACTOR // NS
_ZR = 1024


def _stage_c_body(x1b, e_src, e_dst, agg2b,
                  xg, acc, idx_s, idx_d, rows0, rows1, zbuf,
                  sem0, sem1):
  cid = lax.axis_index("c")
  sid = lax.axis_index("s")

  zeros16 = jnp.zeros((16,), jnp.float32)

  def _zero_row(i, _):
    zbuf[i, pl.ds(0, 16)] = zeros16
    return 0

  lax.fori_loop(0, _ZR, _zero_row, 0)
  a0 = sid * _AR
  x0 = jnp.minimum(sid * _RA, _RA_LAST)

  pltpu.sync_copy(e_src.at[pl.ds(sid * _C2_TILE, _C2_TILE)], idx_s)
  pltpu.sync_copy(e_dst.at[pl.ds(sid * _C2_TILE, _C2_TILE)], idx_d)

  rows = (rows0, rows1)
  sems = (sem0, sem1)

  for j in range(_G_PER_SC):
    g = cid * _G_PER_SC + j

    for k in range(_AR // _ZR):
      pltpu.sync_copy(zbuf, acc.at[pl.ds(a0 + k * _ZR, _ZR)])
    pltpu.sync_copy(x1b.at[g, pl.ds(x0, _RA)], xg.at[pl.ds(x0, _RA)])
    plsc.subcore_barrier()

    pltpu.async_copy(xg.at[idx_s.at[0]], rows0, sem0)

    def _pair(kk, _):
      for b in range(2):
        k = kk * 2 + b

        @pl.when(k < _C2_TILE)
        def _():
          pltpu.make_async_copy(xg.at[idx_s.at[k]], rows[b], sems[b]).wait()

          @pl.when(k + 1 < _C2_TILE)
          def _():
            pltpu.async_copy(xg.at[idx_s.at[k + 1]], rows[1 - b],
                             sems[1 - b])

          pltpu.sync_copy(rows[b], acc.at[idx_d.at[k]], add=True)

      return 0

    lax.fori_loop(0, (_C2_TILE + 1) // 2, _pair, 0)
    plsc.subcore_barrier()

    pltpu.sync_copy(acc.at[pl.ds(a0, _AR)], agg2b.at[g, pl.ds(a0, _AR)])
    plsc.subcore_barrier()


@jax.jit
def _stage_c(x1b, e_src, e_dst):
  return pl.kernel(
      _stage_c_body,
      out_type=jax.ShapeDtypeStruct((NG, N_ACTOR, 16), jnp.float32),
      mesh=_mesh,
      scratch_types=[
          pltpu.VMEM_SHARED((N_TASK, 16), jnp.float32),
          pltpu.VMEM_SHARED((N_ACTOR, 16), jnp.float32),
          pltpu.VMEM((_C2_TILE, CH), jnp.int32),
          pltpu.VMEM((_C2_TILE, CH), jnp.int32),
          pltpu.VMEM((CH, 16), jnp.float32),
          pltpu.VMEM((CH, 16), jnp.float32),
          pltpu.VMEM((_ZR, 16), jnp.float32),
          pltpu.SemaphoreType.DMA,
          pltpu.SemaphoreType.DMA,
      ],
      compiler_params=pltpu.CompilerParams(use_tc_tiling_on_sc=False),
  )(x1b.reshape(NG, N_TASK, 16), e_src.reshape(_C2, CH),
    e_dst.reshape(_C2, CH))


def _mlp1_body(xt, part, w1, b1, w2, b2, out):
  h = xt[...] + part[0] + part[1]
  y = jnp.maximum(jnp.dot(h, w1[...], preferred_element_type=jnp.float32)
                  + b1[...], 0.0)
  x1 = jnp.dot(y, w2[...], preferred_element_type=jnp.float32) + b2[...]
  for g in range(NG):
    out[g] = x1[:, g * 16:(g + 1) * 16]


@jax.jit
def _mlp1(x_task, part, w1, b1, w2, b2):
  blk = 1000
  grid = N_TASK // blk
  return pl.pallas_call(
      _mlp1_body,
      grid=(grid,),
      in_specs=[
          pl.BlockSpec((blk, D), lambda i: (i, 0)),
          pl.BlockSpec((NC, blk, D), lambda i: (0, i, 0)),
          pl.BlockSpec((D, D), lambda i: (0, 0)),
          pl.BlockSpec((1, D), lambda i: (0, 0)),
          pl.BlockSpec((D, D), lambda i: (0, 0)),
          pl.BlockSpec((1, D), lambda i: (0, 0)),
      ],
      out_specs=pl.BlockSpec((NG, blk, 16), lambda i: (0, i, 0)),
      out_shape=jax.ShapeDtypeStruct((NG, N_TASK, 16), jnp.float32),
  )(x_task, part, w1, b1.reshape(1, D), w2, b2.reshape(1, D))


def _mlp2_body(xa, agg, wa1, ba1, wa2, ba2, out):
  h = xa[...] + jnp.concatenate([agg[g] for g in range(NG)], axis=-1)
  y = jnp.maximum(jnp.dot(h, wa1[...], preferred_element_type=jnp.float32)
                  + ba1[...], 0.0)
  out[...] = (jnp.dot(y, wa2[...], preferred_element_type=jnp.float32)
              + ba2[...])


@jax.jit
def _mlp2(x_actor, agg2b, wa1, ba1, wa2, ba2):
  blk = 4096
  grid = N_ACTOR // blk
  return pl.pallas_call(
      _mlp2_body,
      grid=(grid,),
      in_specs=[
          pl.BlockSpec((blk, D), lambda i: (i, 0)),
          pl.BlockSpec((NG, blk, 16), lambda i: (0, i, 0)),
          pl.BlockSpec((D, D), lambda i: (0, 0)),
          pl.BlockSpec((1, D), lambda i: (0, 0)),
          pl.BlockSpec((D, 1), lambda i: (0, 0)),
          pl.BlockSpec((1, 1), lambda i: (0, 0)),
      ],
      out_specs=pl.BlockSpec((blk, 1), lambda i: (i, 0)),
      out_shape=jax.ShapeDtypeStruct((N_ACTOR, 1), jnp.float32),
  )(x_actor, agg2b, wa1, ba1.reshape(1, D), wa2, ba2.reshape(1, 1))


def kernel(x_state, x_task, x_actor, edge_attr_st, W_e, b_e, W1, b1, W2, b2,
           Wa1, ba1, Wa2, ba2, edge_src_st, edge_dst_st, edge_src_ta,
           edge_dst_ta):
  a0 = edge_attr_st[:, 0]
  a1 = edge_attr_st[:, 1]
  part = _stage_a(x_state, a0, a1,
                  edge_src_st.astype(jnp.int32), edge_dst_st.astype(jnp.int32),
                  W_e, b_e)
  x1b = _mlp1(x_task, part, W1, b1, W2, b2)
  agg2b = _stage_c(x1b, edge_src_ta.astype(jnp.int32),
                   edge_dst_ta.astype(jnp.int32))
  logits = _mlp2(x_actor, agg2b, Wa1, ba1, Wa2, ba2)
  return logits.reshape(-1, 64)


# speedup vs baseline: 4.3280x; 1.2049x over previous
import jax
import jax.numpy as jnp
from jax import lax
from jax.experimental import pallas as pl
from jax.experimental.pallas import tpu as pltpu
from jax.experimental.pallas import tpu_sc as plsc

D = 128
N_STATE, N_TASK, N_ACTOR = 50000, 10000, 65536
E1, E2 = 320000, 262144
NC, NS = 2, 16
NW = NC * NS
CH = 128
NG = D // 16

_mesh = plsc.VectorSubcoreMesh(core_axis_name="c", subcore_axis_name="s")


_C1P = 2560
_E1P = _C1P * CH
_CT1 = _C1P // NW
_NTD = N_TASK + 8
_RA = 640
_RA_LAST = N_TASK - _RA
_RAZ_LAST = _NTD - _RA


def _stage_a_body(x_state, a0_h, a1_h, e_src, e_dst, we, be, part,
                  acc, is0, is1, id0, id1, a00, a01, a10, a11, rows0, rows1,
                  wev, bev, gs0, gs1, ls0, ls1):
  cid = lax.axis_index("c")
  sid = lax.axis_index("s")
  wid = sid * NC + cid

  zeros16 = jnp.zeros((16,), jnp.float32)

  def _zero_row(i, _):
    for g in range(NG):
      rows0[i, pl.ds(g * 16, 16)] = zeros16
    return 0

  lax.fori_loop(0, CH, _zero_row, 0)
  rowz = jnp.minimum(sid * _RA, _RAZ_LAST)
  for k in range(_RA // CH):
    pltpu.sync_copy(rows0, acc.at[pl.ds(rowz + k * CH, CH)])

  pltpu.sync_copy(we, wev)
  pltpu.sync_copy(be, bev)
  w0 = [wev[0, pl.ds(g * 16, 16)] for g in range(NG)]
  w1 = [wev[1, pl.ds(g * 16, 16)] for g in range(NG)]
  b = [bev[pl.ds(g * 16, 16)] for g in range(NG)]

  plsc.subcore_barrier()

  rows = (rows0, rows1)
  gsems = (gs0, gs1)
  lsems = (ls0, ls1)
  isrc = (is0, is1)
  idst = (id0, id1)
  a0v = (a00, a01)
  a1v = (a10, a11)
  c0 = wid * _CT1

  def _issue_loads(k, bsel):
    pltpu.async_copy(e_src.at[k + c0], isrc[bsel], lsems[bsel])
    pltpu.async_copy(e_dst.at[k + c0], idst[bsel], lsems[bsel])
    pltpu.async_copy(a0_h.at[k + c0], a0v[bsel], lsems[bsel])
    pltpu.async_copy(a1_h.at[k + c0], a1v[bsel], lsems[bsel])

  def _wait_loads(k, bsel):
    pltpu.make_async_copy(e_src.at[k + c0], isrc[bsel], lsems[bsel]).wait()
    pltpu.make_async_copy(e_dst.at[k + c0], idst[bsel], lsems[bsel]).wait()
    pltpu.make_async_copy(a0_h.at[k + c0], a0v[bsel], lsems[bsel]).wait()
    pltpu.make_async_copy(a1_h.at[k + c0], a1v[bsel], lsems[bsel]).wait()

  _issue_loads(0, 0)
  _wait_loads(0, 0)
  pltpu.async_copy(x_state.at[is0.at[0]], rows0, gs0)
  _issue_loads(1, 1)

  def _pair(kk, _):
    for bsel in range(2):
      k = kk * 2 + bsel
      rb = rows[bsel]

      @pl.when(k + 1 < _CT1)
      def _():
        _wait_loads(k + 1, 1 - bsel)
        pltpu.async_copy(x_state.at[isrc[1 - bsel].at[0]], rows[1 - bsel],
                         gsems[1 - bsel])

      pltpu.make_async_copy(x_state.at[isrc[bsel].at[0]], rb,
                            gsems[bsel]).wait()

      def _grp(j, _):
        av0 = a0v[bsel][0, pl.ds(j * 16, 16)]
        av1 = a1v[bsel][0, pl.ds(j * 16, 16)]
        for i in range(16):
          a0 = jnp.full((16,), av0[i], jnp.float32)
          a1 = jnp.full((16,), av1[i], jnp.float32)
          e = j * 16 + i
          for g in range(NG):
            v = rb[e, pl.ds(g * 16, 16)] + a0 * w0[g] + a1 * w1[g] + b[g]
            rb[e, pl.ds(g * 16, 16)] = jnp.maximum(v, 0.0)
        return 0

      lax.fori_loop(0, CH // 16, _grp, 0)
      pltpu.sync_copy(rb, acc.at[idst[bsel].at[0]], add=True)

      @pl.when(k + 2 < _CT1)
      def _():
        _issue_loads(k + 2, bsel)

    return 0

  lax.fori_loop(0, _CT1 // 2, _pair, 0)
  plsc.subcore_barrier()

  row0 = jnp.minimum(sid * _RA, _RA_LAST)
  pltpu.sync_copy(acc.at[pl.ds(row0, _RA)], part.at[cid, pl.ds(row0, _RA)])


@jax.jit
def _stage_a(x_state, a0_h, a1_h, e_src, e_dst, we, be):
  return pl.kernel(
      _stage_a_body,
      out_type=jax.ShapeDtypeStruct((NC, N_TASK, D), jnp.float32),
      mesh=_mesh,
      scratch_types=[
          pltpu.VMEM_SHARED((_NTD, D), jnp.float32),
          pltpu.VMEM((1, CH), jnp.int32),
          pltpu.VMEM((1, CH), jnp.int32),
          pltpu.VMEM((1, CH), jnp.int32),
          pltpu.VMEM((1, CH), jnp.int32),
          pltpu.VMEM((1, CH), jnp.float32),
          pltpu.VMEM((1, CH), jnp.float32),
          pltpu.VMEM((1, CH), jnp.float32),
          pltpu.VMEM((1, CH), jnp.float32),
          pltpu.VMEM((CH, D), jnp.float32),
          pltpu.VMEM((CH, D), jnp.float32),
          pltpu.VMEM((2, D), jnp.float32),
          pltpu.VMEM((D,), jnp.float32),
          pltpu.SemaphoreType.DMA,
          pltpu.SemaphoreType.DMA,
          pltpu.SemaphoreType.DMA,
          pltpu.SemaphoreType.DMA,
      ],
  )(x_state, a0_h.reshape(_C1P, 1, CH), a1_h.reshape(_C1P, 1, CH),
    e_src.reshape(_C1P, 1, CH), e_dst.reshape(_C1P, 1, CH), we, be)


_C2 = E2 // CH
_C2_TILE = _C2 // NS
_G_PER_SC = NG // NC
_AR = N_ACTOR // NS
_ZR = 1024


def _stage_c_body(x1b, e_src, e_dst, agg2b,
                  xg, acc, idx_s, idx_d, rows0, rows1, zbuf,
                  sem0, sem1):
  cid = lax.axis_index("c")
  sid = lax.axis_index("s")

  zeros16 = jnp.zeros((16,), jnp.float32)

  def _zero_row(i, _):
    zbuf[i, pl.ds(0, 16)] = zeros16
    return 0

  lax.fori_loop(0, _ZR, _zero_row, 0)
  a0 = sid * _AR
  x0 = jnp.minimum(sid * _RA, _RA_LAST)

  pltpu.sync_copy(e_src.at[pl.ds(sid * _C2_TILE, _C2_TILE)], idx_s)
  pltpu.sync_copy(e_dst.at[pl.ds(sid * _C2_TILE, _C2_TILE)], idx_d)

  rows = (rows0, rows1)
  sems = (sem0, sem1)

  for j in range(_G_PER_SC):
    g = cid * _G_PER_SC + j

    for k in range(_AR // _ZR):
      pltpu.sync_copy(zbuf, acc.at[pl.ds(a0 + k * _ZR, _ZR)])
    pltpu.sync_copy(x1b.at[g, pl.ds(x0, _RA)], xg.at[pl.ds(x0, _RA)])
    plsc.subcore_barrier()

    pltpu.async_copy(xg.at[idx_s.at[0]], rows0, sem0)

    def _pair(kk, _):
      for b in range(2):
        k = kk * 2 + b

        @pl.when(k < _C2_TILE)
        def _():
          pltpu.make_async_copy(xg.at[idx_s.at[k]], rows[b], sems[b]).wait()

          @pl.when(k + 1 < _C2_TILE)
          def _():
            pltpu.async_copy(xg.at[idx_s.at[k + 1]], rows[1 - b],
                             sems[1 - b])

          pltpu.sync_copy(rows[b], acc.at[idx_d.at[k]], add=True)

      return 0

    lax.fori_loop(0, (_C2_TILE + 1) // 2, _pair, 0)
    plsc.subcore_barrier()

    pltpu.sync_copy(acc.at[pl.ds(a0, _AR)], agg2b.at[g, pl.ds(a0, _AR)])
    plsc.subcore_barrier()


@jax.jit
def _stage_c(x1b, e_src, e_dst):
  return pl.kernel(
      _stage_c_body,
      out_type=jax.ShapeDtypeStruct((NG, N_ACTOR, 16), jnp.float32),
      mesh=_mesh,
      scratch_types=[
          pltpu.VMEM_SHARED((N_TASK, 16), jnp.float32),
          pltpu.VMEM_SHARED((N_ACTOR, 16), jnp.float32),
          pltpu.VMEM((_C2_TILE, CH), jnp.int32),
          pltpu.VMEM((_C2_TILE, CH), jnp.int32),
          pltpu.VMEM((CH, 16), jnp.float32),
          pltpu.VMEM((CH, 16), jnp.float32),
          pltpu.VMEM((_ZR, 16), jnp.float32),
          pltpu.SemaphoreType.DMA,
          pltpu.SemaphoreType.DMA,
      ],
      compiler_params=pltpu.CompilerParams(use_tc_tiling_on_sc=False),
  )(x1b.reshape(NG, N_TASK, 16), e_src.reshape(_C2, CH),
    e_dst.reshape(_C2, CH))


def _mlp1_body(xt, part, w1, b1, w2, b2, out):
  h = xt[...] + part[0] + part[1]
  y = jnp.maximum(jnp.dot(h, w1[...], preferred_element_type=jnp.float32)
                  + b1[...], 0.0)
  x1 = jnp.dot(y, w2[...], preferred_element_type=jnp.float32) + b2[...]
  for g in range(NG):
    out[g] = x1[:, g * 16:(g + 1) * 16]


@jax.jit
def _mlp1(x_task, part, w1, b1, w2, b2):
  blk = 1000
  grid = N_TASK // blk
  return pl.pallas_call(
      _mlp1_body,
      grid=(grid,),
      in_specs=[
          pl.BlockSpec((blk, D), lambda i: (i, 0)),
          pl.BlockSpec((NC, blk, D), lambda i: (0, i, 0)),
          pl.BlockSpec((D, D), lambda i: (0, 0)),
          pl.BlockSpec((1, D), lambda i: (0, 0)),
          pl.BlockSpec((D, D), lambda i: (0, 0)),
          pl.BlockSpec((1, D), lambda i: (0, 0)),
      ],
      out_specs=pl.BlockSpec((NG, blk, 16), lambda i: (0, i, 0)),
      out_shape=jax.ShapeDtypeStruct((NG, N_TASK, 16), jnp.float32),
  )(x_task, part, w1, b1.reshape(1, D), w2, b2.reshape(1, D))


def _mlp2_body(xa, agg, wa1, ba1, wa2, ba2, out):
  h = xa[...] + jnp.concatenate([agg[g] for g in range(NG)], axis=-1)
  y = jnp.maximum(jnp.dot(h, wa1[...], preferred_element_type=jnp.float32)
                  + ba1[...], 0.0)
  out[...] = (jnp.dot(y, wa2[...], preferred_element_type=jnp.float32)
              + ba2[...])


@jax.jit
def _mlp2(x_actor, agg2b, wa1, ba1, wa2, ba2):
  blk = 4096
  grid = N_ACTOR // blk
  return pl.pallas_call(
      _mlp2_body,
      grid=(grid,),
      in_specs=[
          pl.BlockSpec((blk, D), lambda i: (i, 0)),
          pl.BlockSpec((NG, blk, 16), lambda i: (0, i, 0)),
          pl.BlockSpec((D, D), lambda i: (0, 0)),
          pl.BlockSpec((1, D), lambda i: (0, 0)),
          pl.BlockSpec((D, 1), lambda i: (0, 0)),
          pl.BlockSpec((1, 1), lambda i: (0, 0)),
      ],
      out_specs=pl.BlockSpec((blk, 1), lambda i: (i, 0)),
      out_shape=jax.ShapeDtypeStruct((N_ACTOR, 1), jnp.float32),
  )(x_actor, agg2b, wa1, ba1.reshape(1, D), wa2, ba2.reshape(1, 1))


def kernel(x_state, x_task, x_actor, edge_attr_st, W_e, b_e, W1, b1, W2, b2,
           Wa1, ba1, Wa2, ba2, edge_src_st, edge_dst_st, edge_src_ta,
           edge_dst_ta):
  npad = _E1P - E1
  pad_i = jnp.arange(npad, dtype=jnp.int32)
  a0 = jnp.concatenate([edge_attr_st[:, 0], jnp.zeros((npad,), jnp.float32)])
  a1 = jnp.concatenate([edge_attr_st[:, 1], jnp.zeros((npad,), jnp.float32)])
  src_p = jnp.concatenate([edge_src_st.astype(jnp.int32), pad_i % N_STATE])
  dst_p = jnp.concatenate([edge_dst_st.astype(jnp.int32),
                           N_TASK + (pad_i % 8)])
  part = _stage_a(x_state, a0, a1, src_p, dst_p, W_e, b_e)
  x1b = _mlp1(x_task, part, W1, b1, W2, b2)
  agg2b = _stage_c(x1b, edge_src_ta.astype(jnp.int32),
                   edge_dst_ta.astype(jnp.int32))
  logits = _mlp2(x_actor, agg2b, Wa1, ba1, Wa2, ba2)
  return logits.reshape(-1, 64)
